# baseline (speedup 1.0000x reference)
import functools

import numpy as np

import jax
import jax.numpy as jnp
from jax import lax
from jax.experimental import pallas as pl
from jax.experimental.pallas import tpu as pltpu
from jax.experimental.pallas import tpu_sc as plsc

TARGET_DISC = 1000
SPEED_DISC = 1000
MAX_ROAD_OPTIONS = 10
EMB_DIM = 8
BATCH = 16384

_INFO = plsc.get_sparse_core_info()
_NC, _NS, _L = _INFO.num_cores, _INFO.num_subcores, _INFO.num_lanes
_NW = _NC * _NS
_NROWS = 2 * TARGET_DISC + SPEED_DISC + MAX_ROAD_OPTIONS
_NPAD = 3016
_ROAD_WIN = _NPAD - 1000
_NCOMP = 4
_NGRP = _NW // _NCOMP
_BPW = BATCH // _NGRP
_NVEC = _BPW // _L
_TSEG = 1000


def _f32_linspace(start, stop, num):
    inv = np.float32(np.float32(1.0) / np.float32(num - 1))
    i = np.arange(num - 1, dtype=np.float32)
    t = i * inv
    head = np.float32(start) * (np.float32(1.0) - t) \
        + i * (np.float32(stop) * inv)
    return np.concatenate([head, np.array([stop], np.float32)])


_BINS = np.stack([
    _f32_linspace(-0.001, 0.001, TARGET_DISC),
    _f32_linspace(-0.001, 0.001, TARGET_DISC),
    _f32_linspace(-60.0, 60.0, SPEED_DISC),
    np.zeros(TARGET_DISC, np.float32),
])


def _sc_body(mc_hbm, tab_hbm, bins_hbm, out_hbm,
             m_v, tab_v, bins_v, rows_v, sem):
    wid = lax.axis_index("s") * _NC + lax.axis_index("c")
    comp = lax.rem(wid, _NCOMP)
    base = lax.div(wid, _NCOMP) * _BPW
    is_road = comp == _NCOMP - 1

    with jax.named_scope("in_dma"):
        copies = [
            pltpu.async_copy(mc_hbm.at[comp, pl.ds(base, _BPW)], m_v, sem),
            pltpu.async_copy(bins_hbm.at[comp], bins_v, sem),
        ]
        coff = jnp.where(is_road, _ROAD_WIN, comp * _TSEG)
        for k in range(4):
            p = lax.rem(k + wid, 4) * 2
            copies.append(pltpu.async_copy(
                tab_hbm.at[pl.ds(p, 2), pl.ds(coff, _TSEG)],
                tab_v.at[pl.ds(p, 2)], sem))
        for c in copies:
            c.wait()

    lo = jnp.where(comp < 2, jnp.float32(-0.001),
                   jnp.where(comp == 2, jnp.float32(-60.0), jnp.float32(0.0)))
    inv_step = jnp.where(comp < 2, jnp.float32(499500.0),
                         jnp.where(comp == 2, jnp.float32(8.325),
                                   jnp.float32(1.0)))
    hi_clip = jnp.where(is_road, MAX_ROAD_OPTIONS - 1, _TSEG - 1)
    road16 = jnp.broadcast_to(is_road, (_L,))
    loc_off = jnp.where(is_road, (_NCOMP - 1) * _TSEG - _ROAD_WIN, 0)
    nb = _TSEG

    with jax.named_scope("idx_gather"):
        @plsc.parallel_loop(0, _NVEC, unroll=4)
        def _work(i):
            x = m_v[pl.ds(i * _L, _L)]
            t = jnp.clip((x - lo) * inv_step, -1.0, float(nb)) + 1.0
            g = jnp.clip(lax.convert_element_type(t, jnp.int32), 0, nb)
            bin_hi = plsc.load_gather(bins_v, [jnp.clip(g, 0, nb - 1)])
            bin_lo = plsc.load_gather(bins_v, [jnp.clip(g - 1, 0, nb - 1)])
            one = jnp.full((_L,), 1, jnp.int32)
            zero = jnp.full((_L,), 0, jnp.int32)
            up = jnp.where((g < nb) & (bin_hi <= x), one, zero)
            dn = jnp.where((g > 0) & (bin_lo > x), one, zero)
            srch = g + up - dn
            road = lax.convert_element_type(x, jnp.int32)
            rid = jnp.clip(jnp.where(road16, road, srch), 0, hi_clip) + loc_off
            sl = pl.ds(i * _L, _L)
            for e in range(EMB_DIM):
                row = jnp.full((_L,), e, jnp.int32)
                rows_v[e, sl] = plsc.load_gather(tab_v, [row, rid])

    with jax.named_scope("out_dma"):
        pltpu.sync_copy(
            rows_v,
            out_hbm.at[pl.ds(comp * EMB_DIM, EMB_DIM), pl.ds(base, _BPW)])


_sc_lookup = functools.partial(
    pl.kernel,
    out_type=jax.ShapeDtypeStruct((_NCOMP * EMB_DIM, BATCH), jnp.float32),
    mesh=plsc.VectorSubcoreMesh(core_axis_name="c", subcore_axis_name="s"),
    compiler_params=pltpu.CompilerParams(
        needs_layout_passes=False, use_tc_tiling_on_sc=False),
    scratch_types=[
        pltpu.VMEM((_BPW,), jnp.float32),
        pltpu.VMEM((EMB_DIM, _TSEG), jnp.float32),
        pltpu.VMEM((_TSEG,), jnp.float32),
        pltpu.VMEM((EMB_DIM, _BPW), jnp.float32),
        pltpu.SemaphoreType.DMA,
    ],
)(_sc_body)


def kernel(metrics, target_x_emb, target_y_emb, speed_emb, road_option_emb):
    mcols = metrics.T
    table_t = jnp.concatenate(
        [target_x_emb.T, target_y_emb.T, speed_emb.T, road_option_emb.T,
         jnp.zeros((EMB_DIM, _NPAD - _NROWS), jnp.float32)], axis=1)
    out = _sc_lookup(mcols, table_t, jnp.asarray(_BINS)).T
    return (out, out)
